# baseline (speedup 1.0000x reference)
import functools

import jax
import jax.numpy as jnp
from jax import lax
from jax.experimental import pallas as pl
from jax.experimental.pallas import tpu as pltpu
from jax.experimental.pallas import tpu_sc as plsc

N = 10000
E = 160000
D = 256

NW = 32
NPT = 313
EPW = E // NW
CAP = 5120
L = 16
FC = 32
CCH = 512
HD = D // 2
NEG_INF = float("-inf")


def _ext(v, j):
    return lax.squeeze(lax.slice(v, (j,), (j + 1,)), (0,))


def _bin_edges(src, dst):
    mesh = plsc.VectorSubcoreMesh(core_axis_name="c", subcore_axis_name="s")

    @functools.partial(
        pl.kernel,
        out_type=(
            jax.ShapeDtypeStruct((NW * NW * CAP,), jnp.int32),
            jax.ShapeDtypeStruct((NW * NW * CAP,), jnp.int32),
            jax.ShapeDtypeStruct((NW * NW,), jnp.int32),
        ),
        mesh=mesh,
        scratch_types=[
            pltpu.VMEM((EPW + L,), jnp.int32),
            pltpu.VMEM((EPW + L,), jnp.int32),
            pltpu.VMEM((EPW + L,), jnp.int32),
            pltpu.VMEM((NW * FC,), jnp.int32),
            pltpu.VMEM((NW * FC,), jnp.int32),
            pltpu.VMEM((NW,), jnp.int32),
            pltpu.SMEM((NW,), jnp.int32),
        ],
    )
    def body(src_h, dst_h, bsrc_h, bdst_h, cnt_h, srcv, dstv, tv, stg_s,
             stg_d, cv, cnt):
        wid = lax.axis_index("s") * 2 + lax.axis_index("c")
        ebase = wid * EPW
        lane = lax.iota(jnp.int32, L)
        zeros = jnp.zeros((L,), jnp.int32)

        pltpu.sync_copy(src_h.at[pl.ds(ebase, EPW)], srcv.at[pl.ds(0, EPW)])
        pltpu.sync_copy(dst_h.at[pl.ds(ebase, EPW)], dstv.at[pl.ds(0, EPW)])

        for i in range(NW * FC // L):
            stg_s[pl.ds(i * L, L)] = zeros
            stg_d[pl.ds(i * L, L)] = zeros
        for t in range(NW):
            cnt[t] = 0

        def owner_body(g, _):
            d = dstv[pl.ds(g * L, L)]
            t = jnp.zeros((L,), jnp.int32)
            one = jnp.full((L,), 1, jnp.int32)
            zero = jnp.zeros((L,), jnp.int32)
            for k in range(1, NW):
                t = t + jnp.where(d >= k * NPT, one, zero)
            tv[pl.ds(g * L, L)] = t
            return 0
        lax.fori_loop(0, EPW // L + 1, owner_body, 0)

        def append(dv, sv, tvv, j):
            dj = _ext(dv, j)
            sj = _ext(sv, j)
            tj = _ext(tvv, j)
            c = cnt[tj]
            slot = c & (FC - 1)
            half = slot >> 4
            vaddr = tj * FC + half * L
            pos = slot & (L - 1)
            cur_s = stg_s[pl.ds(vaddr, L)]
            stg_s[pl.ds(vaddr, L)] = jnp.where(lane == pos, sj, cur_s)
            cur_d = stg_d[pl.ds(vaddr, L)]
            stg_d[pl.ds(vaddr, L)] = jnp.where(lane == pos, dj, cur_d)
            cnt[tj] = c + 1

            @pl.when(slot == FC - 1)
            def _():
                bb = (wid * NW + tj) * CAP + pl.multiple_of(c - (FC - 1), FC)
                pltpu.sync_copy(stg_s.at[pl.ds(tj * FC, FC)],
                                bsrc_h.at[pl.ds(bb, FC)])
                pltpu.sync_copy(stg_d.at[pl.ds(tj * FC, FC)],
                                bdst_h.at[pl.ds(bb, FC)])

        def scan_body(g, _):
            dv = dstv[pl.ds(g * L, L)]
            sv = srcv[pl.ds(g * L, L)]
            tvv = tv[pl.ds(g * L, L)]
            for j in range(L):
                append(dv, sv, tvv, j)
            return 0
        lax.fori_loop(0, EPW // L, scan_body, 0)

        ng16 = (EPW // L) * L
        if ng16 < EPW:
            dv = dstv[pl.ds(ng16 - (L - (EPW - ng16)), L)]
            sv = srcv[pl.ds(ng16 - (L - (EPW - ng16)), L)]
            tvv = tv[pl.ds(ng16 - (L - (EPW - ng16)), L)]
            for j in range(L - (EPW - ng16), L):
                append(dv, sv, tvv, j)

        for t in range(NW):
            c = cnt[t]

            @pl.when((c & (FC - 1)) > 0)
            def _():
                bb = (wid * NW + t) * CAP + pl.multiple_of(c & ~(FC - 1), FC)
                pltpu.sync_copy(stg_s.at[pl.ds(t * FC, FC)],
                                bsrc_h.at[pl.ds(bb, FC)])
                pltpu.sync_copy(stg_d.at[pl.ds(t * FC, FC)],
                                bdst_h.at[pl.ds(bb, FC)])

        v0 = jnp.zeros((L,), jnp.int32)
        v1 = jnp.zeros((L,), jnp.int32)
        for t in range(L):
            v0 = jnp.where(lane == t, cnt[t], v0)
            v1 = jnp.where(lane == t, cnt[t + L], v1)
        cv[pl.ds(0, L)] = v0
        cv[pl.ds(L, L)] = v1
        pltpu.sync_copy(cv, cnt_h.at[pl.ds(wid * NW, NW)])

    return body(src, dst)


def _seg_max_sc(table, bsrc, bdst, cnts):
    mesh = plsc.VectorSubcoreMesh(core_axis_name="c", subcore_axis_name="s")

    @functools.partial(
        pl.kernel,
        out_type=jax.ShapeDtypeStruct((2 * N * HD,), jnp.float32),
        mesh=mesh,
        scratch_types=[
            pltpu.VMEM((NPT * HD,), jnp.float32),
            pltpu.VMEM((NPT * HD,), jnp.float32),
            pltpu.VMEM((CAP,), jnp.int32),
            pltpu.VMEM((CAP,), jnp.int32),
            pltpu.VMEM((L, D), jnp.float32),
            pltpu.VMEM((L, D), jnp.float32),
            pltpu.VMEM((L, D), jnp.float32),
            pltpu.VMEM((NW * NW + L,), jnp.int32),
            pltpu.SemaphoreType.DMA,
            pltpu.SemaphoreType.DMA,
            pltpu.SemaphoreType.DMA,
            pltpu.SemaphoreType.DMA,
            pltpu.SemaphoreType.DMA,
        ],
    )
    def body(table_h, bsrc_h, bdst_h, cnt_h, out_h, acc0, acc1, srcv, dstv,
             rows_a, rows_b, rows_c, cntv, sem_a, sem_b, sem_c, sem_d, sem_e):
        t = lax.axis_index("s") * 2 + lax.axis_index("c")
        lo = t * NPT
        neg = jnp.full((L,), NEG_INF, dtype=jnp.float32)

        def init_acc(i, _):
            for u in range(4):
                acc0[pl.ds(i * 4 * L + u * L, L)] = neg
                acc1[pl.ds(i * 4 * L + u * L, L)] = neg
            return 0
        lax.fori_loop(0, NPT * HD // L // 4, init_acc, 0)

        pltpu.sync_copy(cnt_h, cntv.at[pl.ds(0, NW * NW)])

        def rmw(bj, j, rbuf):
            nk = HD // L
            a0 = [acc0[pl.ds(bj + k * L, L)] for k in range(nk)]
            a1 = [acc1[pl.ds(bj + k * L, L)] for k in range(nk)]
            r0 = [rbuf[j, pl.ds(k * L, L)] for k in range(nk)]
            r1 = [rbuf[j, pl.ds(HD + k * L, L)] for k in range(nk)]
            for k in range(nk):
                acc0[pl.ds(bj + k * L, L)] = jnp.maximum(a0[k], r0[k])
            for k in range(nk):
                acc1[pl.ds(bj + k * L, L)] = jnp.maximum(a1[k], r1[k])

        def rmw16(g, rbuf):
            dv = dstv[pl.ds(pl.multiple_of(g * L, L), L)]
            bvec = (dv - lo) * HD
            bjs = [_ext(bvec, j) for j in range(L)]
            for j in range(L):
                rmw(bjs[j], j, rbuf)

        def fire(g, rbuf, sem):
            idx = srcv.at[pl.ds(pl.multiple_of(g * L, L), L)]
            pltpu.async_copy(table_h.at[idx], rbuf, sem)

        def wait(rbuf, sem):
            idx = srcv.at[pl.ds(0, L)]
            pltpu.make_async_copy(table_h.at[idx], rbuf, sem).wait()

        def consume_w(w, _):
            c = _ext(cntv[pl.ds(w * NW + t, L)], 0)
            bb = (w * NW + t) * CAP
            n16 = c >> 4
            rem = c & (L - 1)

            nck = (c + CCH - 1) >> 9

            def cpy(ck, _):
                pltpu.async_copy(bsrc_h.at[pl.ds(bb + ck * CCH, CCH)],
                                 srcv.at[pl.ds(ck * CCH, CCH)], sem_c)
                pltpu.async_copy(bdst_h.at[pl.ds(bb + ck * CCH, CCH)],
                                 dstv.at[pl.ds(ck * CCH, CCH)], sem_d)
                return 0
            lax.fori_loop(0, nck, cpy, 0)

            def cwt(ck, _):
                pltpu.make_async_copy(bsrc_h.at[pl.ds(bb, CCH)],
                                      srcv.at[pl.ds(0, CCH)], sem_c).wait()
                pltpu.make_async_copy(bdst_h.at[pl.ds(bb, CCH)],
                                      dstv.at[pl.ds(0, CCH)], sem_d).wait()
                return 0
            lax.fori_loop(0, nck, cwt, 0)

            noff = pl.multiple_of(n16 * L, L)

            @pl.when(rem > 0)
            def _():
                pltpu.async_copy(
                    table_h.at[srcv.at[pl.ds(noff, L)]], rows_c, sem_e)

            @pl.when(n16 > 0)
            def _():
                fire(0, rows_a, sem_a)

                @pl.when(n16 > 1)
                def _():
                    fire(1, rows_b, sem_b)

                def pbody(p, _):
                    g0 = p * 2
                    wait(rows_a, sem_a)
                    rmw16(g0, rows_a)

                    @pl.when(g0 + 2 < n16)
                    def _():
                        fire(g0 + 2, rows_a, sem_a)
                    wait(rows_b, sem_b)
                    rmw16(g0 + 1, rows_b)

                    @pl.when(g0 + 3 < n16)
                    def _():
                        fire(g0 + 3, rows_b, sem_b)
                    return 0
                lax.fori_loop(0, n16 >> 1, pbody, 0)

                @pl.when((n16 & 1) == 1)
                def _():
                    wait(rows_a, sem_a)
                    rmw16(n16 - 1, rows_a)

            @pl.when(rem > 0)
            def _():
                pltpu.make_async_copy(
                    table_h.at[srcv.at[pl.ds(noff, L)]], rows_c, sem_e
                ).wait()
                dv = dstv[pl.ds(noff, L)]
                bvec = (dv - lo) * HD
                bjs = [_ext(bvec, j) for j in range(L - 1)]
                for j in range(L - 1):
                    @pl.when(j < rem)
                    def _():
                        rmw(bjs[j], j, rows_c)
            return 0

        lax.fori_loop(0, NW, consume_w, 0)

        def fixup(i, _):
            for u in range(4):
                v0 = acc0[pl.ds(i * 4 * L + u * L, L)]
                acc0[pl.ds(i * 4 * L + u * L, L)] = jnp.where(v0 == NEG_INF, 0.0, v0)
                v1 = acc1[pl.ds(i * 4 * L + u * L, L)]
                acc1[pl.ds(i * 4 * L + u * L, L)] = jnp.where(v1 == NEG_INF, 0.0, v1)
            return 0
        lax.fori_loop(0, NPT * HD // L // 4, fixup, 0)

        @pl.when(t < NW - 1)
        def _():
            pltpu.sync_copy(acc0.at[pl.ds(0, NPT * HD)],
                            out_h.at[pl.ds(lo * HD, NPT * HD)])
            pltpu.sync_copy(acc1.at[pl.ds(0, NPT * HD)],
                            out_h.at[pl.ds(N * HD + lo * HD, NPT * HD)])

        @pl.when(t == NW - 1)
        def _():
            last = N - (NW - 1) * NPT
            pltpu.sync_copy(acc0.at[pl.ds(0, last * HD)],
                            out_h.at[pl.ds(lo * HD, last * HD)])
            pltpu.sync_copy(acc1.at[pl.ds(0, last * HD)],
                            out_h.at[pl.ds(N * HD + lo * HD, last * HD)])

    out = body(table, bsrc, bdst, cnts)
    halves = out.reshape(2, N, HD)
    return jnp.concatenate([halves[0], halves[1]], axis=1)


def _mm_body(agg_ref, x_ref, wl_ref, wr_ref, b_ref, o_ref, *, relu):
    y = jnp.dot(agg_ref[...], wl_ref[...], preferred_element_type=jnp.float32)
    y = y + jnp.dot(x_ref[...], wr_ref[...], preferred_element_type=jnp.float32)
    y = y + b_ref[...]
    if relu:
        y = jnp.maximum(y, 0.0)
    o_ref[...] = y


def _sage_dense(agg, x, Wl, Wr, b, relu):
    n, k = x.shape
    h = Wl.shape[0]
    bm = 2000
    grid = (n // bm,)
    return pl.pallas_call(
        functools.partial(_mm_body, relu=relu),
        grid=grid,
        in_specs=[
            pl.BlockSpec((bm, k), lambda i: (i, 0)),
            pl.BlockSpec((bm, k), lambda i: (i, 0)),
            pl.BlockSpec((k, h), lambda i: (0, 0)),
            pl.BlockSpec((k, h), lambda i: (0, 0)),
            pl.BlockSpec((1, h), lambda i: (0, 0)),
        ],
        out_specs=pl.BlockSpec((bm, h), lambda i: (i, 0)),
        out_shape=jax.ShapeDtypeStruct((n, h), jnp.float32),
    )(agg, x, Wl.T, Wr.T, b.reshape(1, h))


def kernel(x, edge_index, Wl1, bl1, Wr1, Wl2, bl2, Wr2, Wl3, bl3, Wr3):
    src = edge_index[0]
    dst = edge_index[1]
    bsrc, bdst, cnts = _bin_edges(src, dst)
    agg1 = _seg_max_sc(x, bsrc, bdst, cnts)
    h1 = _sage_dense(agg1, x, Wl1, Wr1, bl1, relu=True)
    agg2 = _seg_max_sc(h1, bsrc, bdst, cnts)
    h2 = _sage_dense(agg2, h1, Wl2, Wr2, bl2, relu=True)
    agg3 = _seg_max_sc(h2, bsrc, bdst, cnts)
    z = _sage_dense(agg3, h2, Wl3, Wr3, bl3, relu=False)
    return z


# speedup vs baseline: 1.2054x; 1.2054x over previous
import functools

import jax
import jax.numpy as jnp
from jax import lax
from jax.experimental import pallas as pl
from jax.experimental.pallas import tpu as pltpu
from jax.experimental.pallas import tpu_sc as plsc

N = 10000
E = 160000
D = 256

NW = 32
NPT = 313
EPW = E // NW
CAP = 5120
L = 16
FC = 32
CCH = 512
HD = D // 2
NEG_INF = float("-inf")


def _ext(v, j):
    return lax.squeeze(lax.slice(v, (j,), (j + 1,)), (0,))


def _bin_edges(src, dst):
    mesh = plsc.VectorSubcoreMesh(core_axis_name="c", subcore_axis_name="s")

    @functools.partial(
        pl.kernel,
        out_type=(
            jax.ShapeDtypeStruct((NW * NW * CAP,), jnp.int32),
            jax.ShapeDtypeStruct((NW * NW * CAP,), jnp.int32),
            jax.ShapeDtypeStruct((NW * NW,), jnp.int32),
        ),
        mesh=mesh,
        scratch_types=[
            pltpu.VMEM((EPW + L,), jnp.int32),
            pltpu.VMEM((EPW + L,), jnp.int32),
            pltpu.VMEM((EPW + L,), jnp.int32),
            pltpu.VMEM((NW * FC,), jnp.int32),
            pltpu.VMEM((NW * FC,), jnp.int32),
            pltpu.VMEM((NW,), jnp.int32),
            pltpu.SMEM((NW,), jnp.int32),
        ],
    )
    def body(src_h, dst_h, bsrc_h, bdst_h, cnt_h, srcv, dstv, tv, stg_s,
             stg_d, cv, cnt):
        wid = lax.axis_index("s") * 2 + lax.axis_index("c")
        ebase = wid * EPW
        lane = lax.iota(jnp.int32, L)
        zeros = jnp.zeros((L,), jnp.int32)

        pltpu.sync_copy(src_h.at[pl.ds(ebase, EPW)], srcv.at[pl.ds(0, EPW)])
        pltpu.sync_copy(dst_h.at[pl.ds(ebase, EPW)], dstv.at[pl.ds(0, EPW)])

        for i in range(NW * FC // L):
            stg_s[pl.ds(i * L, L)] = zeros
            stg_d[pl.ds(i * L, L)] = zeros
        for t in range(NW):
            cnt[t] = 0

        def owner_body(g, _):
            d = dstv[pl.ds(g * L, L)]
            t = jnp.zeros((L,), jnp.int32)
            one = jnp.full((L,), 1, jnp.int32)
            zero = jnp.zeros((L,), jnp.int32)
            for k in range(1, NW):
                t = t + jnp.where(d >= k * NPT, one, zero)
            tv[pl.ds(g * L, L)] = t
            return 0
        lax.fori_loop(0, EPW // L + 1, owner_body, 0)

        def append(dv, sv, tvv, j):
            dj = _ext(dv, j)
            sj = _ext(sv, j)
            tj = _ext(tvv, j)
            c = cnt[tj]
            slot = c & (FC - 1)
            half = slot >> 4
            vaddr = tj * FC + half * L
            pos = slot & (L - 1)
            cur_s = stg_s[pl.ds(vaddr, L)]
            stg_s[pl.ds(vaddr, L)] = jnp.where(lane == pos, sj, cur_s)
            cur_d = stg_d[pl.ds(vaddr, L)]
            stg_d[pl.ds(vaddr, L)] = jnp.where(lane == pos, dj, cur_d)
            cnt[tj] = c + 1

            @pl.when(slot == FC - 1)
            def _():
                bb = (wid * NW + tj) * CAP + pl.multiple_of(c - (FC - 1), FC)
                pltpu.sync_copy(stg_s.at[pl.ds(tj * FC, FC)],
                                bsrc_h.at[pl.ds(bb, FC)])
                pltpu.sync_copy(stg_d.at[pl.ds(tj * FC, FC)],
                                bdst_h.at[pl.ds(bb, FC)])

        def scan_body(g, _):
            dv = dstv[pl.ds(g * L, L)]
            sv = srcv[pl.ds(g * L, L)]
            tvv = tv[pl.ds(g * L, L)]
            for j in range(L):
                append(dv, sv, tvv, j)
            return 0
        lax.fori_loop(0, EPW // L, scan_body, 0)

        ng16 = (EPW // L) * L
        if ng16 < EPW:
            dv = dstv[pl.ds(ng16 - (L - (EPW - ng16)), L)]
            sv = srcv[pl.ds(ng16 - (L - (EPW - ng16)), L)]
            tvv = tv[pl.ds(ng16 - (L - (EPW - ng16)), L)]
            for j in range(L - (EPW - ng16), L):
                append(dv, sv, tvv, j)

        for t in range(NW):
            c = cnt[t]

            @pl.when((c & (FC - 1)) > 0)
            def _():
                bb = (wid * NW + t) * CAP + pl.multiple_of(c & ~(FC - 1), FC)
                pltpu.sync_copy(stg_s.at[pl.ds(t * FC, FC)],
                                bsrc_h.at[pl.ds(bb, FC)])
                pltpu.sync_copy(stg_d.at[pl.ds(t * FC, FC)],
                                bdst_h.at[pl.ds(bb, FC)])

        v0 = jnp.zeros((L,), jnp.int32)
        v1 = jnp.zeros((L,), jnp.int32)
        for t in range(L):
            v0 = jnp.where(lane == t, cnt[t], v0)
            v1 = jnp.where(lane == t, cnt[t + L], v1)
        cv[pl.ds(0, L)] = v0
        cv[pl.ds(L, L)] = v1
        pltpu.sync_copy(cv, cnt_h.at[pl.ds(wid * NW, NW)])

    return body(src, dst)


def _seg_max_sc(table, bsrc, bdst, cnts):
    mesh = plsc.VectorSubcoreMesh(core_axis_name="c", subcore_axis_name="s")

    @functools.partial(
        pl.kernel,
        out_type=jax.ShapeDtypeStruct((2 * N * HD,), jnp.float32),
        mesh=mesh,
        scratch_types=[
            pltpu.VMEM((NPT * HD,), jnp.float32),
            pltpu.VMEM((NPT * HD,), jnp.float32),
            pltpu.VMEM((2 * CAP,), jnp.int32),
            pltpu.VMEM((2 * CAP,), jnp.int32),
            pltpu.VMEM((L, D), jnp.float32),
            pltpu.VMEM((L, D), jnp.float32),
            pltpu.VMEM((L, D), jnp.float32),
            pltpu.VMEM((NW * NW + L,), jnp.int32),
            pltpu.SemaphoreType.DMA,
            pltpu.SemaphoreType.DMA,
            pltpu.SemaphoreType.DMA,
            pltpu.SemaphoreType.DMA,
            pltpu.SemaphoreType.DMA,
        ],
    )
    def body(table_h, bsrc_h, bdst_h, cnt_h, out_h, acc0, acc1, srcv, dstv,
             rows_a, rows_b, rows_c, cntv, sem_a, sem_b, sem_c, sem_d, sem_e):
        t = lax.axis_index("s") * 2 + lax.axis_index("c")
        lo = t * NPT
        neg = jnp.full((L,), NEG_INF, dtype=jnp.float32)

        def init_acc(i, _):
            for u in range(4):
                acc0[pl.ds(i * 4 * L + u * L, L)] = neg
                acc1[pl.ds(i * 4 * L + u * L, L)] = neg
            return 0
        lax.fori_loop(0, NPT * HD // L // 4, init_acc, 0)

        pltpu.sync_copy(cnt_h, cntv.at[pl.ds(0, NW * NW)])

        def fire_idx(w, poff):
            cw = _ext(cntv[pl.ds(w * NW + t, L)], 0)
            bbw = (w * NW + t) * CAP

            def cpy(ck, _):
                pltpu.async_copy(bsrc_h.at[pl.ds(bbw + ck * CCH, CCH)],
                                 srcv.at[pl.ds(poff + ck * CCH, CCH)], sem_c)
                pltpu.async_copy(bdst_h.at[pl.ds(bbw + ck * CCH, CCH)],
                                 dstv.at[pl.ds(poff + ck * CCH, CCH)], sem_d)
                return 0
            lax.fori_loop(0, (cw + CCH - 1) >> 9, cpy, 0)

        def rmw(bvec, j, rbuf):
            bj = _ext(bvec, j)
            nk = HD // L
            a0 = [acc0[pl.ds(bj + k * L, L)] for k in range(nk)]
            a1 = [acc1[pl.ds(bj + k * L, L)] for k in range(nk)]
            r0 = [rbuf[j, pl.ds(k * L, L)] for k in range(nk)]
            r1 = [rbuf[j, pl.ds(HD + k * L, L)] for k in range(nk)]
            for k in range(nk):
                acc0[pl.ds(bj + k * L, L)] = jnp.maximum(a0[k], r0[k])
            for k in range(nk):
                acc1[pl.ds(bj + k * L, L)] = jnp.maximum(a1[k], r1[k])

        def rmw16(g, poff, rbuf):
            dv = dstv[pl.ds(poff + pl.multiple_of(g * L, L), L)]
            bvec = (dv - lo) * HD
            for j in range(L):
                rmw(bvec, j, rbuf)

        def fire(g, poff, rbuf, sem):
            idx = srcv.at[pl.ds(poff + pl.multiple_of(g * L, L), L)]
            pltpu.async_copy(table_h.at[idx], rbuf, sem)

        def wait(rbuf, sem):
            idx = srcv.at[pl.ds(0, L)]
            pltpu.make_async_copy(table_h.at[idx], rbuf, sem).wait()

        def consume_w(w, _):
            c = _ext(cntv[pl.ds(w * NW + t, L)], 0)
            bb = (w * NW + t) * CAP
            poff = pl.multiple_of((w & 1) * CAP, CCH)
            n16 = c >> 4
            rem = c & (L - 1)

            nck = (c + CCH - 1) >> 9

            def cwt(ck, _):
                pltpu.make_async_copy(bsrc_h.at[pl.ds(bb, CCH)],
                                      srcv.at[pl.ds(0, CCH)], sem_c).wait()
                pltpu.make_async_copy(bdst_h.at[pl.ds(bb, CCH)],
                                      dstv.at[pl.ds(0, CCH)], sem_d).wait()
                return 0
            lax.fori_loop(0, nck, cwt, 0)

            @pl.when(w + 1 < NW)
            def _():
                fire_idx(w + 1, pl.multiple_of((1 - (w & 1)) * CAP, CCH))

            noff = pl.multiple_of(n16 * L, L) + poff

            @pl.when(rem > 0)
            def _():
                pltpu.async_copy(
                    table_h.at[srcv.at[pl.ds(noff, L)]], rows_c, sem_e)

            @pl.when(n16 > 0)
            def _():
                fire(0, poff, rows_a, sem_a)

                @pl.when(n16 > 1)
                def _():
                    fire(1, poff, rows_b, sem_b)

                def pbody(p, _):
                    g0 = p * 2
                    wait(rows_a, sem_a)
                    rmw16(g0, poff, rows_a)

                    @pl.when(g0 + 2 < n16)
                    def _():
                        fire(g0 + 2, poff, rows_a, sem_a)
                    wait(rows_b, sem_b)
                    rmw16(g0 + 1, poff, rows_b)

                    @pl.when(g0 + 3 < n16)
                    def _():
                        fire(g0 + 3, poff, rows_b, sem_b)
                    return 0
                lax.fori_loop(0, n16 >> 1, pbody, 0)

                @pl.when((n16 & 1) == 1)
                def _():
                    wait(rows_a, sem_a)
                    rmw16(n16 - 1, poff, rows_a)

            @pl.when(rem > 0)
            def _():
                pltpu.make_async_copy(
                    table_h.at[srcv.at[pl.ds(noff, L)]], rows_c, sem_e
                ).wait()
                dv = dstv[pl.ds(noff, L)]
                bvec = (dv - lo) * HD
                for j in range(L - 1):
                    @pl.when(j < rem)
                    def _():
                        rmw(bvec, j, rows_c)
            return 0

        fire_idx(0, 0)
        lax.fori_loop(0, NW, consume_w, 0)

        def fixup(i, _):
            for u in range(4):
                v0 = acc0[pl.ds(i * 4 * L + u * L, L)]
                acc0[pl.ds(i * 4 * L + u * L, L)] = jnp.where(v0 == NEG_INF, 0.0, v0)
                v1 = acc1[pl.ds(i * 4 * L + u * L, L)]
                acc1[pl.ds(i * 4 * L + u * L, L)] = jnp.where(v1 == NEG_INF, 0.0, v1)
            return 0
        lax.fori_loop(0, NPT * HD // L // 4, fixup, 0)

        @pl.when(t < NW - 1)
        def _():
            pltpu.sync_copy(acc0.at[pl.ds(0, NPT * HD)],
                            out_h.at[pl.ds(lo * HD, NPT * HD)])
            pltpu.sync_copy(acc1.at[pl.ds(0, NPT * HD)],
                            out_h.at[pl.ds(N * HD + lo * HD, NPT * HD)])

        @pl.when(t == NW - 1)
        def _():
            last = N - (NW - 1) * NPT
            pltpu.sync_copy(acc0.at[pl.ds(0, last * HD)],
                            out_h.at[pl.ds(lo * HD, last * HD)])
            pltpu.sync_copy(acc1.at[pl.ds(0, last * HD)],
                            out_h.at[pl.ds(N * HD + lo * HD, last * HD)])

    out = body(table, bsrc, bdst, cnts)
    halves = out.reshape(2, N, HD)
    return jnp.concatenate([halves[0], halves[1]], axis=1)


def _mm_body(agg_ref, x_ref, wl_ref, wr_ref, b_ref, o_ref, *, relu):
    y = jnp.dot(agg_ref[...], wl_ref[...], preferred_element_type=jnp.float32)
    y = y + jnp.dot(x_ref[...], wr_ref[...], preferred_element_type=jnp.float32)
    y = y + b_ref[...]
    if relu:
        y = jnp.maximum(y, 0.0)
    o_ref[...] = y


def _sage_dense(agg, x, Wl, Wr, b, relu):
    n, k = x.shape
    h = Wl.shape[0]
    bm = 2000
    grid = (n // bm,)
    return pl.pallas_call(
        functools.partial(_mm_body, relu=relu),
        grid=grid,
        in_specs=[
            pl.BlockSpec((bm, k), lambda i: (i, 0)),
            pl.BlockSpec((bm, k), lambda i: (i, 0)),
            pl.BlockSpec((k, h), lambda i: (0, 0)),
            pl.BlockSpec((k, h), lambda i: (0, 0)),
            pl.BlockSpec((1, h), lambda i: (0, 0)),
        ],
        out_specs=pl.BlockSpec((bm, h), lambda i: (i, 0)),
        out_shape=jax.ShapeDtypeStruct((n, h), jnp.float32),
    )(agg, x, Wl.T, Wr.T, b.reshape(1, h))


def kernel(x, edge_index, Wl1, bl1, Wr1, Wl2, bl2, Wr2, Wl3, bl3, Wr3):
    src = edge_index[0]
    dst = edge_index[1]
    bsrc, bdst, cnts = _bin_edges(src, dst)
    agg1 = _seg_max_sc(x, bsrc, bdst, cnts)
    h1 = _sage_dense(agg1, x, Wl1, Wr1, bl1, relu=True)
    agg2 = _seg_max_sc(h1, bsrc, bdst, cnts)
    h2 = _sage_dense(agg2, h1, Wl2, Wr2, bl2, relu=True)
    agg3 = _seg_max_sc(h2, bsrc, bdst, cnts)
    z = _sage_dense(agg3, h2, Wl3, Wr3, bl3, relu=False)
    return z
